# baseline (speedup 1.0000x reference)
import functools

import jax
import jax.numpy as jnp
from jax import lax
from jax.experimental import pallas as pl
from jax.experimental.pallas import tpu as pltpu
from jax.experimental.pallas import tpu_sc as plsc

B, L, D = 4096, 200, 64
WINDOW = 4
LANES = 16
DC = D // LANES

NC, NS = 2, 16
NW = NC * NS
ROWS_PER_W = B // NW
R = 4
CHUNK_T = R * L
N_CHUNKS = ROWS_PER_W // R
SUB_OFFS = list(range(0, CHUNK_T - CHUNK_T % 128, 128))
SUB_SIZES = [128] * len(SUB_OFFS)
if CHUNK_T % 128:
    SUB_OFFS.append(CHUNK_T - CHUNK_T % 128)
    SUB_SIZES.append(CHUNK_T % 128)


def _sc_body(tok_hbm, emb_hbm, out_hbm, idx_v, in_v, out_v, sem):
    wid = lax.axis_index("c") * NS + lax.axis_index("s")

    def chunk_body(ci, _):
        base = pl.multiple_of(
            wid * (ROWS_PER_W * L) + ci * CHUNK_T, 8
        )
        pltpu.sync_copy(tok_hbm.at[pl.ds(base, CHUNK_T)], idx_v)
        cps = [
            pltpu.async_copy(
                emb_hbm.at[idx_v.at[pl.ds(off, sz)]],
                in_v.at[pl.ds(off, sz)],
                sem,
            )
            for off, sz in zip(SUB_OFFS, SUB_SIZES)
        ]
        for cp in cps:
            cp.wait()

        zero = jnp.zeros((LANES,), jnp.float32)
        for r in range(R):
            def lbody(l, carry, r=r):
                row = r * L + l
                new = []
                for dc in range(DC):
                    s1, s2, s3 = carry[dc]
                    e = in_v[row, pl.ds(dc * LANES, LANES)]
                    o = s3 + e
                    out_v[row, pl.ds(dc * LANES, LANES)] = o
                    ne = o - s3
                    new.append((ne, s1 + ne, s2 + ne))
                return tuple(new)

            init = tuple((zero, zero, zero) for _ in range(DC))
            lax.fori_loop(0, L, lbody, init)

        pltpu.sync_copy(out_v, out_hbm.at[pl.ds(base, CHUNK_T)])
        return 0

    lax.fori_loop(0, N_CHUNKS, chunk_body, 0)


@jax.jit
def _sc_call(tok2d, embedding):
    mesh = plsc.VectorSubcoreMesh(
        core_axis_name="c", subcore_axis_name="s", num_cores=NC, num_subcores=NS
    )
    f = pl.kernel(
        _sc_body,
        out_type=jax.ShapeDtypeStruct((B * L, D), jnp.float32),
        mesh=mesh,
        scratch_types=[
            pltpu.VMEM((CHUNK_T,), jnp.int32),
            pltpu.VMEM((CHUNK_T, D), jnp.float32),
            pltpu.VMEM((CHUNK_T, D), jnp.float32),
            pltpu.SemaphoreType.DMA,
        ],
        compiler_params=pltpu.CompilerParams(use_tc_tiling_on_sc=False),
    )
    return f(tok2d, embedding)


def kernel(tokens, embedding):
    tok = tokens.reshape(-1).astype(jnp.int32)
    out = _sc_call(tok, embedding)
    return out.reshape(B, L, D)


# speedup vs baseline: 1.1152x; 1.1152x over previous
import functools

import jax
import jax.numpy as jnp
from jax import lax
from jax.experimental import pallas as pl
from jax.experimental.pallas import tpu as pltpu
from jax.experimental.pallas import tpu_sc as plsc

B, L, D = 4096, 200, 64
WINDOW = 4
LANES = 16
DC = D // LANES
UNROLL = 4

NC, NS = 2, 16
NW = NC * NS
ROWS_PER_W = B // NW
TOK_PER_W = ROWS_PER_W * L
R = 2
CHUNK_T = R * L
N_CH = ROWS_PER_W // R
NBUF = 4
SUBS = [(o, min(128, CHUNK_T - o)) for o in range(0, CHUNK_T, 128)]


def _sc_body(tok_hbm, emb_hbm, out_hbm, idx_all, b0, b1, b2, b3, gsem, wsem):
    bufs = (b0, b1, b2, b3)
    wid = lax.axis_index("c") * NS + lax.axis_index("s")
    wbase = pl.multiple_of(wid * TOK_PER_W, 8)
    pltpu.sync_copy(tok_hbm.at[pl.ds(wbase, TOK_PER_W)], idx_all)

    def fire_gather(c, b):
        ioff = pl.multiple_of(c * CHUNK_T, 8)
        for off, sz in SUBS:
            pltpu.async_copy(
                emb_hbm.at[idx_all.at[pl.ds(ioff + off, sz)]],
                bufs[b].at[pl.ds(off, sz)],
                gsem.at[b],
            )

    def wait_gather(b):
        pltpu.make_async_copy(
            emb_hbm.at[pl.ds(0, CHUNK_T)], bufs[b], gsem.at[b]
        ).wait()

    def fire_wb(c, b):
        off = pl.multiple_of(wbase + c * CHUNK_T, 8)
        pltpu.async_copy(bufs[b], out_hbm.at[pl.ds(off, CHUNK_T)], wsem.at[b])

    def wait_wb(b):
        pltpu.make_async_copy(
            bufs[b], out_hbm.at[pl.ds(0, CHUNK_T)], wsem.at[b]
        ).wait()

    def compute(b):
        buf = bufs[b]
        zero = jnp.zeros((LANES,), jnp.float32)
        for r in range(R):
            def lbody(k, carry, r=r):
                cur = carry
                for u in range(UNROLL):
                    row = r * L + k * UNROLL + u
                    nxt = []
                    for dc in range(DC):
                        s1, s2, s3 = cur[dc]
                        e = buf[row, pl.ds(dc * LANES, LANES)]
                        o = s3 + e
                        buf[row, pl.ds(dc * LANES, LANES)] = o
                        ne = o - s3
                        nxt.append((ne, s1 + ne, s2 + ne))
                    cur = tuple(nxt)
                return cur

            init = tuple((zero, zero, zero) for _ in range(DC))
            lax.fori_loop(0, L // UNROLL, lbody, init)

    for c in range(NBUF - 1):
        fire_gather(c, c)

    def iter_body(i, _):
        for j in range(NBUF):
            c = i * NBUF + j
            wait_gather(j)
            compute(j)
            fire_wb(c, j)
            d = c + NBUF - 1
            bd = (j + NBUF - 1) % NBUF
            if j == 0:
                @pl.when(i > 0)
                def _():
                    wait_wb(bd)

                fire_gather(d, bd)
            else:
                wait_wb(bd)

                @pl.when(i < N_CH // NBUF - 1)
                def _():
                    fire_gather(d, bd)
        return 0

    lax.fori_loop(0, N_CH // NBUF, iter_body, 0)
    wait_wb(NBUF - 1)


@jax.jit
def _sc_call(tok, embedding):
    mesh = plsc.VectorSubcoreMesh(
        core_axis_name="c", subcore_axis_name="s", num_cores=NC, num_subcores=NS
    )
    f = pl.kernel(
        _sc_body,
        out_type=jax.ShapeDtypeStruct((B * L, D), jnp.float32),
        mesh=mesh,
        scratch_types=[
            pltpu.VMEM((TOK_PER_W,), jnp.int32),
            pltpu.VMEM((CHUNK_T, D), jnp.float32),
            pltpu.VMEM((CHUNK_T, D), jnp.float32),
            pltpu.VMEM((CHUNK_T, D), jnp.float32),
            pltpu.VMEM((CHUNK_T, D), jnp.float32),
            pltpu.SemaphoreType.DMA((NBUF,)),
            pltpu.SemaphoreType.DMA((NBUF,)),
        ],
        compiler_params=pltpu.CompilerParams(use_tc_tiling_on_sc=False),
    )
    return f(tok, embedding)


def kernel(tokens, embedding):
    tok = tokens.reshape(-1).astype(jnp.int32)
    out = _sc_call(tok, embedding)
    return out.reshape(B, L, D)
